# baseline (speedup 1.0000x reference)
import jax
import jax.numpy as jnp
from jax import lax
from jax.experimental import pallas as pl
from jax.experimental.pallas import tpu as pltpu
from jax.experimental.pallas import tpu_sc as plsc

B, L, D = 4096, 50, 128
N = B * L
NF = 5
OD = NF * D
NC, NS = 2, 16
NW = NC * NS
TPW = N // NW
C = 32
G = C // 16
NCHUNK = TPW // C
NROWS = (26, 128, 128, 128, 128)
OFFS = (0, 26, 154, 282, 410)
TROWS = 538


def _sc_body(*refs):
  (l_h, x_h, y_h, w_h, h_h,
   lt_h, xt_h, yt_h, wt_h, ht_h,
   out_h,
   tab_v,
   i00, i01, i02, i03, i04,
   i10, i11, i12, i13, i14,
   rows0, rows1,
   is0, is1, ss0, ss1) = refs

  idx_hbms = (l_h, x_h, y_h, w_h, h_h)
  tab_hbms = (lt_h, xt_h, yt_h, wt_h, ht_h)
  idxbuf = ((i00, i01, i02, i03, i04), (i10, i11, i12, i13, i14))
  rows = (rows0, rows1)
  isem = (is0, is1)
  ssem = (ss0, ss1)

  wid = lax.axis_index("s") * NC + lax.axis_index("c")
  base = wid * TPW

  for f in range(NF):
    pltpu.sync_copy(tab_hbms[f], tab_v.at[pl.ds(OFFS[f] * D, NROWS[f] * D)])

  def fire_idx(ci, b):
    for f in range(NF):
      pltpu.async_copy(idx_hbms[f].at[pl.ds(base + ci * C, C)],
                       idxbuf[b][f], isem[b])

  def wait_idx(ci, b):
    for f in range(NF):
      pltpu.make_async_copy(idx_hbms[f].at[pl.ds(base + ci * C, C)],
                            idxbuf[b][f], isem[b]).wait()

  def fire_store(ci, b):
    pltpu.async_copy(rows[b], out_h.at[pl.ds((base + ci * C) * OD, C * OD)],
                     ssem[b])

  def wait_store(ci, b):
    pltpu.make_async_copy(rows[b],
                          out_h.at[pl.ds((base + ci * C) * OD, C * OD)],
                          ssem[b]).wait()

  def compute(b):
    bases = []
    for g in range(G):
      tokv = lax.broadcasted_iota(jnp.int32, (16,), 0) + (g * 16)
      tokbase = tokv * OD
      for f in range(NF):
        idxv = idxbuf[b][f][pl.ds(g * 16, 16)]
        rowbase = (idxv + OFFS[f]) * D
        bases.append((rowbase, tokbase + f * D))

    def step(j):
      for rowbase, outbase in bases:
        vals = plsc.load_gather(tab_v, [rowbase + j])
        plsc.store_scatter(rows[b], [outbase + j], vals)

    pl.loop(0, D)(step)

  fire_idx(0, 0)
  wait_idx(0, 0)
  fire_idx(1, 1)
  compute(0)
  fire_store(0, 0)
  wait_idx(1, 1)
  fire_idx(2, 0)
  compute(1)
  fire_store(1, 1)

  def gstep(gi):
    g2 = gi * 2
    for b in (0, 1):
      ci = g2 + b
      wait_idx(ci, b)

      @pl.when(ci + 1 < NCHUNK)
      def _():
        fire_idx(ci + 1, 1 - b)

      wait_store(ci - 2, b)
      compute(b)
      fire_store(ci, b)

  pl.loop(1, NCHUNK // 2)(gstep)

  wait_store(NCHUNK - 2, 0)
  wait_store(NCHUNK - 1, 1)


@jax.jit
def kernel(label, x, y, w, h, label_table, x_table, y_table, w_table, h_table):
  idx = [jnp.swapaxes(a, 0, 1).reshape(N).astype(jnp.int32)
         for a in (label, x, y, w, h)]
  tabs = [t.reshape(-1) for t in
          (label_table, x_table, y_table, w_table, h_table)]
  mesh = plsc.VectorSubcoreMesh(core_axis_name="c", subcore_axis_name="s",
                                num_cores=NC, num_subcores=NS)
  run = pl.kernel(
      _sc_body,
      out_type=jax.ShapeDtypeStruct((N * OD,), jnp.float32),
      mesh=mesh,
      compiler_params=pltpu.CompilerParams(needs_layout_passes=False),
      scratch_types=(
          [pltpu.VMEM((TROWS * D,), jnp.float32)]
          + [pltpu.VMEM((C,), jnp.int32) for _ in range(2 * NF)]
          + [pltpu.VMEM((C * OD,), jnp.float32) for _ in range(2)]
          + [pltpu.SemaphoreType.DMA for _ in range(4)]
      ),
  )
  out = run(*idx, *tabs)
  return jnp.swapaxes(out.reshape(L, B, OD), 0, 1)


# speedup vs baseline: 1.9409x; 1.9409x over previous
import jax
import jax.numpy as jnp
from jax import lax
from jax.experimental import pallas as pl
from jax.experimental.pallas import tpu as pltpu
from jax.experimental.pallas import tpu_sc as plsc

B, L, D = 4096, 50, 128
N = B * L
NF = 5
OD = NF * D
ODP = OD + 1
NC, NS = 2, 16
NW = NC * NS
TPW = N // NW
C = 32
G = C // 16
NCHUNK = TPW // C
NROWS = (26, 128, 128, 128, 128)
TOFF = (0, 26 * D, 154 * D, 282 * D, 410 * D)
TWORDS = 538 * D


def _sc_body(*refs):
  (l_h, x_h, y_h, w_h, h_h,
   lt_h, xt_h, yt_h, wt_h, ht_h,
   out_h,
   tab_v,
   i00, i01, i02, i03, i04,
   i10, i11, i12, i13, i14,
   rows0, rows1,
   is0, is1, ss0, ss1) = refs

  idx_hbms = (l_h, x_h, y_h, w_h, h_h)
  tab_hbms = (lt_h, xt_h, yt_h, wt_h, ht_h)
  idxbuf = ((i00, i01, i02, i03, i04), (i10, i11, i12, i13, i14))
  rows = (rows0, rows1)
  isem = (is0, is1)
  ssem = (ss0, ss1)

  wid = lax.axis_index("s") * NC + lax.axis_index("c")
  base = wid * TPW

  for f in range(NF):
    pltpu.sync_copy(tab_hbms[f], tab_v.at[pl.ds(TOFF[f], NROWS[f] * D)])

  def fire_idx(ci, b):
    for f in range(NF):
      pltpu.async_copy(idx_hbms[f].at[pl.ds(base + ci * C, C)],
                       idxbuf[b][f], isem[b])

  def wait_idx(ci, b):
    for f in range(NF):
      pltpu.make_async_copy(idx_hbms[f].at[pl.ds(base + ci * C, C)],
                            idxbuf[b][f], isem[b]).wait()

  def fire_store(ci, b):
    pltpu.async_copy(rows[b].at[:, pl.ds(0, OD)],
                     out_h.at[pl.ds(base + ci * C, C)], ssem[b])

  def wait_store(ci, b):
    pltpu.make_async_copy(rows[b].at[:, pl.ds(0, OD)],
                          out_h.at[pl.ds(base + ci * C, C)], ssem[b]).wait()

  def compute(b):
    bases = []
    for g in range(G):
      tokv = lax.broadcasted_iota(jnp.int32, (16,), 0) + (g * 16)
      for f in range(NF):
        idxv = idxbuf[b][f][pl.ds(g * 16, 16)]
        bases.append((f, idxv + TOFF[f], tokv, jnp.full((16,), f * D,
                                                        jnp.int32)))

    def step(j):
      for f, gbase, tokv, colv in bases:
        vals = plsc.load_gather(tab_v, [gbase + j * NROWS[f]])
        plsc.store_scatter(rows[b], [tokv, colv + j], vals)

    pl.loop(0, D)(step)

  fire_idx(0, 0)
  wait_idx(0, 0)
  fire_idx(1, 1)
  compute(0)
  fire_store(0, 0)
  wait_idx(1, 1)
  fire_idx(2, 0)
  compute(1)
  fire_store(1, 1)

  def gstep(gi):
    g2 = gi * 2
    for b in (0, 1):
      ci = g2 + b
      wait_idx(ci, b)

      @pl.when(ci + 1 < NCHUNK)
      def _():
        fire_idx(ci + 1, 1 - b)

      wait_store(ci - 2, b)
      compute(b)
      fire_store(ci, b)

  pl.loop(1, NCHUNK // 2)(gstep)

  wait_store(NCHUNK - 2, 0)
  wait_store(NCHUNK - 1, 1)


@jax.jit
def kernel(label, x, y, w, h, label_table, x_table, y_table, w_table, h_table):
  idx = [jnp.swapaxes(a, 0, 1).reshape(N).astype(jnp.int32)
         for a in (label, x, y, w, h)]
  tabs = [t.T.reshape(-1) for t in
          (label_table, x_table, y_table, w_table, h_table)]
  mesh = plsc.VectorSubcoreMesh(core_axis_name="c", subcore_axis_name="s",
                                num_cores=NC, num_subcores=NS)
  run = pl.kernel(
      _sc_body,
      out_type=jax.ShapeDtypeStruct((N, OD), jnp.float32),
      mesh=mesh,
      compiler_params=pltpu.CompilerParams(needs_layout_passes=False),
      scratch_types=(
          [pltpu.VMEM((TWORDS,), jnp.float32)]
          + [pltpu.VMEM((C,), jnp.int32) for _ in range(2 * NF)]
          + [pltpu.VMEM((C, ODP), jnp.float32) for _ in range(2)]
          + [pltpu.SemaphoreType.DMA for _ in range(4)]
      ),
  )
  out = run(*idx, *tabs)
  return jnp.swapaxes(out.reshape(L, B, OD), 0, 1)


# speedup vs baseline: 4.7488x; 2.4467x over previous
import jax
import jax.numpy as jnp
from jax import lax
from jax.experimental import pallas as pl
from jax.experimental.pallas import tpu as pltpu
from jax.experimental.pallas import tpu_sc as plsc

B, L, D = 4096, 50, 128
N = B * L
NF = 5
OD = NF * D
NC, NS = 2, 16
NW = NC * NS
TPW = N // NW
C = 40
NCHUNK = TPW // C
NB = 4


def _sc_body(*refs):
  (l_h, x_h, y_h, w_h, h_h,
   lt_h, xt_h, yt_h, wt_h, ht_h,
   out_h) = refs[:11]
  idxbuf = tuple(refs[11 + 5 * b:11 + 5 * (b + 1)] for b in range(NB))
  rows = refs[31:35]
  isem = refs[35:39]
  gsem = refs[39:43]
  ssem = refs[43:47]

  idx_hbms = (l_h, x_h, y_h, w_h, h_h)
  tab_hbms = (lt_h, xt_h, yt_h, wt_h, ht_h)

  wid = lax.axis_index("s") * NC + lax.axis_index("c")
  base = wid * TPW

  def fire_idx(ci, b):
    for f in range(NF):
      pltpu.async_copy(idx_hbms[f].at[pl.ds(base + ci * C, C)],
                       idxbuf[b][f], isem[b])

  def wait_idx(ci, b):
    for f in range(NF):
      pltpu.make_async_copy(idx_hbms[f].at[pl.ds(base + ci * C, C)],
                            idxbuf[b][f], isem[b]).wait()

  def fire_gathers(ci, b):
    for f in range(NF):
      pltpu.async_copy(tab_hbms[f].at[idxbuf[b][f]],
                       rows[b].at[:, pl.ds(f * D, D)], gsem[b])

  def wait_gathers(ci, b):
    for f in range(NF):
      pltpu.make_async_copy(tab_hbms[f].at[idxbuf[b][f]],
                            rows[b].at[:, pl.ds(f * D, D)], gsem[b]).wait()

  def fire_store(ci, b):
    pltpu.async_copy(rows[b], out_h.at[pl.ds(base + ci * C, C)], ssem[b])

  def wait_store(ci, b):
    pltpu.make_async_copy(rows[b], out_h.at[pl.ds(base + ci * C, C)],
                          ssem[b]).wait()

  for ci in range(NB):
    fire_idx(ci, ci)
  wait_idx(0, 0)
  fire_gathers(0, 0)
  wait_idx(1, 1)
  fire_gathers(1, 1)
  wait_idx(2, 2)
  fire_gathers(2, 2)
  wait_gathers(0, 0)
  fire_idx(NB, 0)
  fire_store(0, 0)
  wait_idx(3, 3)
  fire_gathers(3, 3)
  wait_gathers(1, 1)
  fire_idx(NB + 1, 1)
  fire_store(1, 1)

  def gstep(gi):
    g4 = gi * NB
    for b in range(NB):
      ci = g4 + b
      wait_idx(ci, b)
      wait_store(ci - NB, b)
      fire_gathers(ci, b)
      b2 = (b - 2) % NB
      wait_gathers(ci - 2, b2)

      @pl.when(ci + 2 < NCHUNK)
      def _():
        fire_idx(ci + 2, b2)

      fire_store(ci - 2, b2)

  pl.loop(1, NCHUNK // NB)(gstep)

  wait_gathers(NCHUNK - 2, (NCHUNK - 2) % NB)
  fire_store(NCHUNK - 2, (NCHUNK - 2) % NB)
  wait_gathers(NCHUNK - 1, (NCHUNK - 1) % NB)
  fire_store(NCHUNK - 1, (NCHUNK - 1) % NB)
  for k in range(NB):
    wait_store(NCHUNK - NB + k, (NCHUNK - NB + k) % NB)


@jax.jit
def kernel(label, x, y, w, h, label_table, x_table, y_table, w_table, h_table):
  idx = [jnp.swapaxes(a, 0, 1).reshape(N).astype(jnp.int32)
         for a in (label, x, y, w, h)]
  mesh = plsc.VectorSubcoreMesh(core_axis_name="c", subcore_axis_name="s",
                                num_cores=NC, num_subcores=NS)
  run = pl.kernel(
      _sc_body,
      out_type=jax.ShapeDtypeStruct((N, OD), jnp.float32),
      mesh=mesh,
      scratch_types=(
          [pltpu.VMEM((C,), jnp.int32) for _ in range(NB * NF)]
          + [pltpu.VMEM((C, OD), jnp.float32) for _ in range(NB)]
          + [pltpu.SemaphoreType.DMA for _ in range(3 * NB)]
      ),
  )
  out = run(*idx, label_table, x_table, y_table, w_table, h_table)
  return jnp.swapaxes(out.reshape(L, B, OD), 0, 1)


# speedup vs baseline: 7.4675x; 1.5725x over previous
import jax
import jax.numpy as jnp
from jax import lax
from jax.experimental import pallas as pl
from jax.experimental.pallas import tpu as pltpu
from jax.experimental.pallas import tpu_sc as plsc

B, L, D = 4096, 50, 128
N = B * L
NF = 5
NSF = 4
OD = NF * D
ODP = OD + 1
NC, NS = 2, 16
NW = NC * NS
TPW = N // NW
C = 32
G = C // 16
NCHUNK = TPW // C
NB = 4
LROWS = 26


def _sc_body(*refs):
  (l_h, x_h, y_h, w_h, h_h,
   lt_h, xt_h, yt_h, wt_h, ht_h,
   out_h, ltab_v) = refs[:12]
  idxbuf = tuple(refs[12 + 5 * b:12 + 5 * (b + 1)] for b in range(NB))
  rows = refs[32:36]
  isem = refs[36:40]
  gsem = refs[40:44]
  ssem = refs[44:48]

  idx_hbms = (l_h, x_h, y_h, w_h, h_h)
  geo_hbms = (xt_h, yt_h, wt_h, ht_h)

  wid = lax.axis_index("s") * NC + lax.axis_index("c")
  base = wid * TPW

  pltpu.sync_copy(lt_h, ltab_v)

  def fire_idx(ci, b):
    for f in range(NF):
      pltpu.async_copy(idx_hbms[f].at[pl.ds(base + ci * C, C)],
                       idxbuf[b][f], isem[b])

  def wait_idx(ci, b):
    for f in range(NF):
      pltpu.make_async_copy(idx_hbms[f].at[pl.ds(base + ci * C, C)],
                            idxbuf[b][f], isem[b]).wait()

  def fire_gathers(ci, b):
    for f in range(NSF):
      pltpu.async_copy(geo_hbms[f].at[idxbuf[b][f + 1]],
                       rows[b].at[:, pl.ds((f + 1) * D, D)], gsem[b])

  def wait_gathers(ci, b):
    for f in range(NSF):
      pltpu.make_async_copy(geo_hbms[f].at[idxbuf[b][f + 1]],
                            rows[b].at[:, pl.ds((f + 1) * D, D)],
                            gsem[b]).wait()

  def fire_store(ci, b):
    pltpu.async_copy(rows[b].at[:, pl.ds(0, OD)],
                     out_h.at[pl.ds(base + ci * C, C)], ssem[b])

  def wait_store(ci, b):
    pltpu.make_async_copy(rows[b].at[:, pl.ds(0, OD)],
                          out_h.at[pl.ds(base + ci * C, C)], ssem[b]).wait()

  def label_gather(b):
    groups = []
    for g in range(G):
      tokv = lax.broadcasted_iota(jnp.int32, (16,), 0) + (g * 16)
      idxv = idxbuf[b][0][pl.ds(g * 16, 16)]
      groups.append((idxv, tokv, jnp.zeros((16,), jnp.int32)))

    def step(j):
      for idxv, tokv, zero in groups:
        vals = plsc.load_gather(ltab_v, [idxv + j * LROWS])
        plsc.store_scatter(rows[b], [tokv, zero + j], vals)

    pl.loop(0, D)(step)

  for ci in range(NB):
    fire_idx(ci, ci)
  wait_idx(0, 0)
  fire_gathers(0, 0)
  label_gather(0)
  wait_idx(1, 1)
  fire_gathers(1, 1)
  label_gather(1)
  wait_idx(2, 2)
  fire_gathers(2, 2)
  label_gather(2)
  wait_gathers(0, 0)
  fire_idx(NB, 0)
  fire_store(0, 0)
  wait_idx(3, 3)
  fire_gathers(3, 3)
  label_gather(3)
  wait_gathers(1, 1)
  fire_idx(NB + 1, 1)
  fire_store(1, 1)

  def gstep(gi):
    g4 = gi * NB
    for b in range(NB):
      ci = g4 + b
      wait_idx(ci, b)
      wait_store(ci - NB, b)
      fire_gathers(ci, b)
      label_gather(b)
      b2 = (b - 2) % NB
      wait_gathers(ci - 2, b2)

      @pl.when(ci + 2 < NCHUNK)
      def _():
        fire_idx(ci + 2, b2)

      fire_store(ci - 2, b2)

  pl.loop(1, NCHUNK // NB)(gstep)

  wait_gathers(NCHUNK - 2, (NCHUNK - 2) % NB)
  fire_store(NCHUNK - 2, (NCHUNK - 2) % NB)
  wait_gathers(NCHUNK - 1, (NCHUNK - 1) % NB)
  fire_store(NCHUNK - 1, (NCHUNK - 1) % NB)
  for k in range(NB):
    wait_store(NCHUNK - NB + k, (NCHUNK - NB + k) % NB)


@jax.jit
def kernel(label, x, y, w, h, label_table, x_table, y_table, w_table, h_table):
  idx = [jnp.swapaxes(a, 0, 1).reshape(N).astype(jnp.int32)
         for a in (label, x, y, w, h)]
  lt_t = label_table.T.reshape(-1)
  mesh = plsc.VectorSubcoreMesh(core_axis_name="c", subcore_axis_name="s",
                                num_cores=NC, num_subcores=NS)
  run = pl.kernel(
      _sc_body,
      out_type=jax.ShapeDtypeStruct((N, OD), jnp.float32),
      mesh=mesh,
      compiler_params=pltpu.CompilerParams(needs_layout_passes=False),
      scratch_types=(
          [pltpu.VMEM((LROWS * D,), jnp.float32)]
          + [pltpu.VMEM((C,), jnp.int32) for _ in range(NB * NF)]
          + [pltpu.VMEM((C, ODP), jnp.float32) for _ in range(NB)]
          + [pltpu.SemaphoreType.DMA for _ in range(3 * NB)]
      ),
  )
  out = run(*idx, lt_t, x_table, y_table, w_table, h_table)
  return jnp.swapaxes(out.reshape(L, B, OD), 0, 1)
